# Initial kernel scaffold; baseline (speedup 1.0000x reference)
import jax
import jax.numpy as jnp
from jax.experimental import pallas as pl


def kernel(heatmap):
    raise NotImplementedError("write your pallas kernel here")


# speedup vs baseline: 11.1072x; 11.1072x over previous
import jax
import jax.numpy as jnp
from jax import lax
from jax.experimental import pallas as pl
from jax.experimental.pallas import tpu as pltpu

_H = 512
_W = 512
_R = 9
_OUT = _H - _R + 1
_NT = 8
_BIG = 1 << 30
_SCALE = 1.0 / 81.0


def _pool_rows(win, nrows):
    vs = win[0:nrows, :]
    for k in range(1, _R):
        vs = vs + win[k:k + nrows, :]
    agg = vs[:, 0:_OUT]
    for k in range(1, _R):
        agg = agg + vs[:, k:k + _OUT]
    return agg * _SCALE


def _row_stats(agg, nrows):
    rmax = jnp.max(agg, axis=1, keepdims=True)
    cio = lax.broadcasted_iota(jnp.int32, (nrows, _OUT), 1)
    rcol = jnp.min(jnp.where(agg == rmax, cio, _BIG), axis=1, keepdims=True)
    return rmax, rcol


def _tc_body(x_ref, out_ref, hm_s, rmax_s, rcode_s):
    x = x_ref[0, 0]
    hm_s[:, :] = x

    agg = _pool_rows(x, _OUT)
    rmax, rcol = _row_stats(agg, _OUT)
    rowio = lax.broadcasted_iota(jnp.int32, (_OUT, 1), 0)
    rcode = rowio * 1024 + rcol
    rmax_s[:, :] = jnp.concatenate(
        [jnp.broadcast_to(rmax, (_OUT, 128)),
         jnp.full((_H - _OUT, 128), -jnp.inf, jnp.float32)], axis=0)
    rcode_s[:, :] = jnp.concatenate(
        [jnp.broadcast_to(rcode, (_OUT, 128)),
         jnp.full((_H - _OUT, 128), _BIG, jnp.int32)], axis=0)

    for t in range(_NT):
        blk = rmax_s[:, :]
        m = jnp.max(blk)
        code = jnp.min(jnp.where(blk == m, rcode_s[:, :], _BIG))
        r0 = code >> 10
        c0 = code & 1023
        out_ref[0, t, 0] = c0 + 4
        out_ref[0, t, 1] = r0 + 4

        s2 = (jnp.minimum(jnp.maximum(r0 - 8, 0), 480) // 8) * 8
        win = hm_s[pl.ds(s2, 32), :]
        grow = lax.broadcasted_iota(jnp.int32, (32, _W), 0) + s2
        gcol = lax.broadcasted_iota(jnp.int32, (32, _W), 1)
        msk = (grow >= r0) & (grow < r0 + _R) & (gcol >= c0) & (gcol < c0 + _R)
        win = jnp.where(msk, 0.0, win)
        hm_s[pl.ds(s2, 32), :] = win

        aggl = _pool_rows(win, 24)
        rml, rcl = _row_stats(aggl, 24)
        rio24 = lax.broadcasted_iota(jnp.int32, (24, 1), 0) + s2
        rmax_s[pl.ds(s2, 24), :] = jnp.broadcast_to(rml, (24, 128))
        rcode_s[pl.ds(s2, 24), :] = jnp.broadcast_to(rio24 * 1024 + rcl, (24, 128))


def kernel(heatmap):
    B = heatmap.shape[0]
    return pl.pallas_call(
        _tc_body,
        grid=(B,),
        in_specs=[pl.BlockSpec((1, 1, _H, _W), lambda b: (b, 0, 0, 0))],
        out_specs=pl.BlockSpec((1, _NT, 2), lambda b: (b, 0, 0),
                               memory_space=pltpu.SMEM),
        out_shape=jax.ShapeDtypeStruct((B, _NT, 2), jnp.int32),
        scratch_shapes=[
            pltpu.VMEM((_H, _W), jnp.float32),
            pltpu.VMEM((_H, 128), jnp.float32),
            pltpu.VMEM((_H, 128), jnp.int32),
        ],
    )(heatmap)
